# Initial kernel scaffold; baseline (speedup 1.0000x reference)
import functools

import jax
import jax.numpy as jnp
from jax import lax
from jax.experimental import pallas as pl
from jax.experimental.pallas import tpu as pltpu
from jax.experimental.pallas import tpu_sc as plsc

L = 16
NUM_ROWS = 17
UNROLL = 8


def _hist_partials(cards, table, nc, ns):
    nw = nc * ns
    n = cards.shape[0]
    per_w = n // nw

    mesh = plsc.VectorSubcoreMesh(core_axis_name="c", subcore_axis_name="s")

    @functools.partial(
        pl.kernel,
        mesh=mesh,
        out_type=jax.ShapeDtypeStruct((nw, L), jnp.float32),
        scratch_types=[
            pltpu.VMEM((per_w,), jnp.int32),
            pltpu.VMEM((NUM_ROWS, L), jnp.int32),
            pltpu.VMEM((NUM_ROWS, L), jnp.float32),
            pltpu.VMEM((L,), jnp.float32),
        ],
    )
    def hist_kernel(cards_hbm, table_hbm, out_hbm, cards_v, counts_v, table_v, partial_v):
        wid = lax.axis_index("s") * nc + lax.axis_index("c")
        base = wid * per_w
        pltpu.sync_copy(cards_hbm.at[pl.ds(base, per_w)], cards_v)
        pltpu.sync_copy(table_hbm, table_v)

        zeros16 = jnp.zeros((L,), jnp.int32)
        for r in range(NUM_ROWS):
            counts_v[r, :] = zeros16

        lane = lax.iota(jnp.int32, L)
        ones = jnp.ones((L,), jnp.int32)

        def body(i, carry):
            b = i * (UNROLL * L)
            for u in range(UNROLL):
                c = cards_v[pl.ds(b + u * L, L)]
                rank_row = c % 13
                suit_row = 13 + c // 13
                plsc.addupdate_scatter(counts_v, [rank_row, lane], ones)
                plsc.addupdate_scatter(counts_v, [suit_row, lane], ones)
            return carry

        lax.fori_loop(0, per_w // (UNROLL * L), body, 0)

        acc = jnp.zeros((L,), jnp.float32)
        for r in range(NUM_ROWS):
            tot = jnp.sum(counts_v[r, :])
            acc = acc + tot.astype(jnp.float32) * table_v[r, :]
        partial_v[...] = acc
        pltpu.sync_copy(partial_v, out_hbm.at[wid])

    return hist_kernel(cards, table)


def kernel(cards, rank_embed, suit_embed):
    n = cards.shape[0]
    info = plsc.get_sparse_core_info()
    nc, ns = info.num_cores, info.num_subcores

    table = jnp.zeros((NUM_ROWS, L), jnp.float32)
    table = table.at[:13, :8].set(rank_embed)
    table = table.at[13:, 8:12].set(suit_embed)

    partials = _hist_partials(cards, table, nc, ns)
    return (partials.sum(axis=0) / n)[:12]


# speedup vs baseline: 55.0787x; 55.0787x over previous
import functools

import jax
import jax.numpy as jnp
from jax import lax
from jax.experimental import pallas as pl
from jax.experimental.pallas import tpu as pltpu
from jax.experimental.pallas import tpu_sc as plsc

L = 16
NUM_ROWS = 17
UNROLL = 8


def _hist_partials(cards, table, nc, ns):
    nw = nc * ns
    n = cards.shape[0]
    per_w = n // nw

    mesh = plsc.VectorSubcoreMesh(core_axis_name="c", subcore_axis_name="s")

    @functools.partial(
        pl.kernel,
        mesh=mesh,
        compiler_params=pltpu.CompilerParams(needs_layout_passes=False),
        out_type=jax.ShapeDtypeStruct((nw, L), jnp.float32),
        scratch_types=[
            pltpu.VMEM((per_w,), jnp.int32),
            pltpu.VMEM((NUM_ROWS, L), jnp.int32),
            pltpu.VMEM((NUM_ROWS, L), jnp.float32),
            pltpu.VMEM((L,), jnp.float32),
        ],
    )
    def hist_kernel(cards_hbm, table_hbm, out_hbm, cards_v, counts_v, table_v, partial_v):
        wid = lax.axis_index("s") * nc + lax.axis_index("c")
        base = wid * per_w
        pltpu.sync_copy(cards_hbm.at[pl.ds(base, per_w)], cards_v)
        pltpu.sync_copy(table_hbm, table_v)

        zeros16 = jnp.zeros((L,), jnp.int32)
        for r in range(NUM_ROWS):
            counts_v[r, :] = zeros16

        lane = lax.iota(jnp.int32, L)
        ones = jnp.ones((L,), jnp.int32)

        def body(i, carry):
            b = i * (UNROLL * L)
            for u in range(UNROLL):
                c = cards_v[pl.ds(b + u * L, L)]
                rank_row = c % 13
                suit_row = 13 + c // 13
                plsc.addupdate_scatter(counts_v, [rank_row, lane], ones)
                plsc.addupdate_scatter(counts_v, [suit_row, lane], ones)
            return carry

        lax.fori_loop(0, per_w // (UNROLL * L), body, 0)

        acc = jnp.zeros((L,), jnp.float32)
        for r in range(NUM_ROWS):
            tot = jnp.sum(counts_v[r, :])
            acc = acc + tot.astype(jnp.float32) * table_v[r, :]
        partial_v[...] = acc
        pltpu.sync_copy(partial_v, out_hbm.at[wid])

    return hist_kernel(cards, table)


def kernel(cards, rank_embed, suit_embed):
    n = cards.shape[0]
    info = plsc.get_sparse_core_info()
    nc, ns = info.num_cores, info.num_subcores

    table = jnp.zeros((NUM_ROWS, L), jnp.float32)
    table = table.at[:13, :8].set(rank_embed)
    table = table.at[13:, 8:12].set(suit_embed)

    partials = _hist_partials(cards, table, nc, ns)
    return (partials.sum(axis=0) / n)[:12]


# speedup vs baseline: 155.3054x; 2.8197x over previous
import functools

import jax
import jax.numpy as jnp
from jax import lax
from jax.experimental import pallas as pl
from jax.experimental.pallas import tpu as pltpu
from jax.experimental.pallas import tpu_sc as plsc

L = 16
NUM_ROWS = 52
UNROLL = 16


def _hist_partials(cards, table, nc, ns):
    nw = nc * ns
    n = cards.shape[0]
    per_w = n // nw

    mesh = plsc.VectorSubcoreMesh(core_axis_name="c", subcore_axis_name="s")

    @functools.partial(
        pl.kernel,
        mesh=mesh,
        compiler_params=pltpu.CompilerParams(needs_layout_passes=False),
        out_type=jax.ShapeDtypeStruct((nw, L), jnp.float32),
        scratch_types=[
            pltpu.VMEM((per_w,), jnp.int32),
            pltpu.VMEM((NUM_ROWS, L), jnp.int32),
            pltpu.VMEM((NUM_ROWS, L), jnp.float32),
            pltpu.VMEM((L,), jnp.float32),
        ],
    )
    def hist_kernel(cards_hbm, table_hbm, out_hbm, cards_v, counts_v, table_v, partial_v):
        wid = lax.axis_index("s") * nc + lax.axis_index("c")
        base = wid * per_w
        pltpu.sync_copy(cards_hbm.at[pl.ds(base, per_w)], cards_v)
        pltpu.sync_copy(table_hbm, table_v)

        zeros16 = jnp.zeros((L,), jnp.int32)
        for r in range(NUM_ROWS):
            counts_v[r, :] = zeros16

        lane = lax.iota(jnp.int32, L)
        ones = jnp.ones((L,), jnp.int32)

        def body(i, carry):
            b = i * (UNROLL * L)
            for u in range(UNROLL):
                c = cards_v[pl.ds(b + u * L, L)]
                plsc.addupdate_scatter(counts_v, [c, lane], ones)
            return carry

        lax.fori_loop(0, per_w // (UNROLL * L), body, 0)

        acc = jnp.zeros((L,), jnp.float32)
        for r in range(NUM_ROWS):
            tot = jnp.sum(counts_v[r, :])
            acc = acc + tot.astype(jnp.float32) * table_v[r, :]
        partial_v[...] = acc
        pltpu.sync_copy(partial_v, out_hbm.at[wid])

    return hist_kernel(cards, table)


def kernel(cards, rank_embed, suit_embed):
    n = cards.shape[0]
    info = plsc.get_sparse_core_info()
    nc, ns = info.num_cores, info.num_subcores

    card_ids = jnp.arange(NUM_ROWS, dtype=jnp.int32)
    table = jnp.concatenate(
        [
            jnp.take(rank_embed, card_ids % 13, axis=0),
            jnp.take(suit_embed, card_ids // 13, axis=0),
            jnp.zeros((NUM_ROWS, 4), jnp.float32),
        ],
        axis=1,
    )

    partials = _hist_partials(cards, table, nc, ns)
    return (partials.sum(axis=0) / n)[:12]


# speedup vs baseline: 208.5078x; 1.3426x over previous
import functools

import jax
import jax.numpy as jnp
from jax import lax
from jax.experimental import pallas as pl
from jax.experimental.pallas import tpu as pltpu
from jax.experimental.pallas import tpu_sc as plsc

L = 16
NUM_ROWS = 52
UNROLL = 16


def _hist_partials(cards, table, nc, ns):
    nw = nc * ns
    n = cards.shape[0]
    per_w = n // nw

    mesh = plsc.VectorSubcoreMesh(core_axis_name="c", subcore_axis_name="s")

    @functools.partial(
        pl.kernel,
        mesh=mesh,
        compiler_params=pltpu.CompilerParams(needs_layout_passes=False),
        out_type=jax.ShapeDtypeStruct((nw, L), jnp.float32),
        scratch_types=[
            pltpu.VMEM((per_w,), jnp.int32),
            pltpu.VMEM((NUM_ROWS, L), jnp.int32),
            pltpu.VMEM((NUM_ROWS, L), jnp.float32),
            pltpu.VMEM((L,), jnp.float32),
        ],
    )
    def hist_kernel(cards_hbm, table_hbm, out_hbm, cards_v, counts_v, table_v, partial_v):
        wid = lax.axis_index("s") * nc + lax.axis_index("c")
        base = wid * per_w
        pltpu.sync_copy(cards_hbm.at[pl.ds(base, per_w)], cards_v)
        pltpu.sync_copy(table_hbm, table_v)

        zeros16 = jnp.zeros((L,), jnp.int32)
        for r in range(NUM_ROWS):
            counts_v[r, :] = zeros16

        lane = lax.iota(jnp.int32, L)
        ones = jnp.ones((L,), jnp.int32)

        @plsc.parallel_loop(0, per_w // L, step=1, unroll=UNROLL)
        def body(j):
            c = cards_v[pl.ds(j * L, L)]
            plsc.addupdate_scatter(counts_v, [c, lane], ones)

        acc = jnp.zeros((L,), jnp.float32)
        for r in range(NUM_ROWS):
            tot = jnp.sum(counts_v[r, :])
            acc = acc + tot.astype(jnp.float32) * table_v[r, :]
        partial_v[...] = acc
        pltpu.sync_copy(partial_v, out_hbm.at[wid])

    return hist_kernel(cards, table)


def kernel(cards, rank_embed, suit_embed):
    n = cards.shape[0]
    info = plsc.get_sparse_core_info()
    nc, ns = info.num_cores, info.num_subcores

    card_ids = jnp.arange(NUM_ROWS, dtype=jnp.int32)
    table = jnp.concatenate(
        [
            jnp.take(rank_embed, card_ids % 13, axis=0),
            jnp.take(suit_embed, card_ids // 13, axis=0),
            jnp.zeros((NUM_ROWS, 4), jnp.float32),
        ],
        axis=1,
    )

    partials = _hist_partials(cards, table, nc, ns)
    return (partials.sum(axis=0) / n)[:12]


# speedup vs baseline: 215.6498x; 1.0343x over previous
import functools

import jax
import jax.numpy as jnp
from jax import lax
from jax.experimental import pallas as pl
from jax.experimental.pallas import tpu as pltpu
from jax.experimental.pallas import tpu_sc as plsc

L = 16
NUM_BINS = 52
UNROLL = 16


def _mean_partials(cards, rank_embed, suit_embed, nc, ns):
    nw = nc * ns
    n = cards.shape[0]
    per_w = n // nw
    half = per_w // 2
    inv_n = 1.0 / n

    mesh = plsc.VectorSubcoreMesh(core_axis_name="c", subcore_axis_name="s")

    @functools.partial(
        pl.kernel,
        mesh=mesh,
        compiler_params=pltpu.CompilerParams(needs_layout_passes=False),
        out_type=jax.ShapeDtypeStruct((nw, L), jnp.float32),
        scratch_types=[
            pltpu.VMEM((per_w,), jnp.int32),
            pltpu.VMEM((NUM_BINS * L,), jnp.int32),
            pltpu.VMEM((13, 8), jnp.float32),
            pltpu.VMEM((4, 4), jnp.float32),
            pltpu.VMEM((L,), jnp.float32),
            pltpu.SemaphoreType.DMA,
            pltpu.SemaphoreType.DMA,
            pltpu.SemaphoreType.DMA,
            pltpu.SemaphoreType.DMA,
        ],
    )
    def hist_kernel(cards_hbm, rank_hbm, suit_hbm, out_hbm, cards_v, counts_v,
                    rank_v, suit_v, partial_v, sem0, sem1, sem2, sem3):
        wid = lax.axis_index("s") * nc + lax.axis_index("c")
        base = wid * per_w
        cp0 = pltpu.async_copy(cards_hbm.at[pl.ds(base, half)],
                               cards_v.at[pl.ds(0, half)], sem0)
        cp1 = pltpu.async_copy(cards_hbm.at[pl.ds(base + half, half)],
                               cards_v.at[pl.ds(half, half)], sem1)
        cpr = pltpu.async_copy(rank_hbm, rank_v, sem2)
        cps = pltpu.async_copy(suit_hbm, suit_v, sem3)

        zeros16 = jnp.zeros((L,), jnp.int32)
        for r in range(NUM_BINS):
            counts_v[pl.ds(r * L, L)] = zeros16

        lane = lax.iota(jnp.int32, L)
        lane_base = lane * NUM_BINS
        ones = jnp.ones((L,), jnp.int32)

        cp0.wait()

        @plsc.parallel_loop(0, half // L, step=1, unroll=UNROLL)
        def body0(j):
            c = cards_v[pl.ds(j * L, L)]
            plsc.addupdate_scatter(counts_v, [lane_base + c], ones)

        cp1.wait()

        @plsc.parallel_loop(half // L, per_w // L, step=1, unroll=UNROLL)
        def body1(j):
            c = cards_v[pl.ds(j * L, L)]
            plsc.addupdate_scatter(counts_v, [lane_base + c], ones)

        cpr.wait()
        cps.wait()

        lane7 = lane & 7
        lane3 = lane & 3
        rmask = jnp.where(lane < 8, 1.0, 0.0).astype(jnp.float32)
        smask = jnp.where((lane >= 8) & (lane < 12), 1.0, 0.0).astype(jnp.float32)

        acc = jnp.zeros((L,), jnp.float32)
        for c in range(NUM_BINS):
            cnt = plsc.load_gather(counts_v, [lane_base + c])
            tot = jnp.sum(cnt).astype(jnp.float32)
            rrow = plsc.load_gather(rank_v, [jnp.full((L,), c % 13, jnp.int32), lane7])
            srow = plsc.load_gather(suit_v, [jnp.full((L,), c // 13, jnp.int32), lane3])
            acc = acc + tot * (rrow * rmask + srow * smask)
        partial_v[...] = acc * inv_n
        pltpu.sync_copy(partial_v, out_hbm.at[wid])

    return hist_kernel(cards, rank_embed, suit_embed)


def kernel(cards, rank_embed, suit_embed):
    info = plsc.get_sparse_core_info()
    nc, ns = info.num_cores, info.num_subcores
    partials = _mean_partials(cards, rank_embed, suit_embed, nc, ns)
    return partials.sum(axis=0)[:12]
